# baseline (speedup 1.0000x reference)
import jax
import jax.numpy as jnp
from jax import lax
from jax.experimental import pallas as pl
from jax.experimental.pallas import tpu as pltpu

_PROJ_TS = 4


def _lstm_kernel(x0_ref, xn_ref, wih_ref, whh_ref, b_ref, wfc_ref, bfc_ref,
                 out_ref, h_sc, c_sc, ga_sc, gb_sc, rg_sc):
    chunk = pl.program_id(1)
    n_chunks = pl.num_programs(1)
    tc, Bt, D = xn_ref.shape
    H = h_sc.shape[1]
    G = ga_sc.shape[2]
    Bh = Bt // 2

    @pl.when(chunk == 0)
    def _init():
        h_sc[...] = jnp.zeros_like(h_sc)
        c_sc[...] = jnp.zeros_like(c_sc)
        rg_sc[...] = jnp.zeros_like(rg_sc)
        x0 = x0_ref[...].reshape(tc * Bt, D)
        ga_sc[...] = ((jnp.dot(x0, wih_ref[...],
                               preferred_element_type=jnp.float32)
                       + b_ref[...])
                      .reshape(tc, Bt, G).astype(ga_sc.dtype))

    def _half(lo, hi, gx, c):
        gates = gx.astype(jnp.float32) + rg_sc[lo:hi, :].astype(jnp.float32)
        sig = 0.5 * jnp.tanh(gates[:, :3 * H]) + 0.5
        i = sig[:, 0 * H:1 * H]
        f = sig[:, 1 * H:2 * H]
        o = sig[:, 2 * H:3 * H]
        g = jnp.tanh(gates[:, 3 * H:])
        c_new = f * c + i * g
        h_new = o * jnp.tanh(c_new)
        rg_sc[lo:hi, :] = jnp.dot(h_new.astype(whh_ref.dtype), whh_ref[...],
                                  preferred_element_type=jnp.float32
                                  ).astype(rg_sc.dtype)
        return h_new, c_new

    def _phase(src, dst):
        nq = 2
        Bq = Bt // nq
        cs = [c_sc[q * Bq:(q + 1) * Bq, :] for q in range(nq)]
        hs = [None] * nq
        for t0 in range(0, tc, _PROJ_TS):
            xn = xn_ref[t0:t0 + _PROJ_TS].reshape(_PROJ_TS * Bt, D)
            gn = (jnp.dot(xn, wih_ref[...],
                          preferred_element_type=jnp.float32)
                  + b_ref[...]).astype(dst.dtype)
            dst[t0:t0 + _PROJ_TS] = gn.reshape(_PROJ_TS, Bt, G)

            for k in range(_PROJ_TS):
                gx = src[t0 + k]
                for q in range(nq):
                    hs[q], cs[q] = _half(q * Bq, (q + 1) * Bq,
                                         gx[q * Bq:(q + 1) * Bq, :], cs[q])
        for q in range(nq):
            c_sc[q * Bq:(q + 1) * Bq, :] = cs[q]
            h_sc[q * Bq:(q + 1) * Bq, :] = hs[q]

    parity = lax.rem(chunk, 2)

    @pl.when(parity == 0)
    def _even():
        _phase(ga_sc, gb_sc)

    @pl.when(parity == 1)
    def _odd():
        _phase(gb_sc, ga_sc)

    @pl.when(chunk == n_chunks - 1)
    def _fc():
        out_ref[...] = (jnp.dot(h_sc[...].astype(wfc_ref.dtype), wfc_ref[...],
                                preferred_element_type=jnp.float32)
                        + bfc_ref[...]).astype(out_ref.dtype)


def _permute_ifgo_to_ifog(w):
    i, f, g, o = jnp.split(w, 4, axis=0)
    return jnp.concatenate([i, f, o, g], axis=0)


def _scale_sigmoid_cols(w, H):
    return jnp.concatenate([0.5 * w[:3 * H], w[3 * H:]], axis=0)


def kernel(x, w_ih, w_hh, b_ih, b_hh, w_fc, b_fc):
    B, T, D = x.shape
    H = w_hh.shape[1]
    C = w_fc.shape[0]
    G = 4 * H

    b_tile = B // 2
    t_chunk = 16
    n_chunks = T // t_chunk
    mm_dtype = jnp.bfloat16

    x_tm = jnp.transpose(x, (1, 0, 2)).astype(mm_dtype)
    wih_p = _scale_sigmoid_cols(_permute_ifgo_to_ifog(w_ih), H)
    whh_p = _scale_sigmoid_cols(_permute_ifgo_to_ifog(w_hh), H)
    b_p = _scale_sigmoid_cols(_permute_ifgo_to_ifog(b_ih + b_hh), H)
    wih_t = jnp.transpose(wih_p).astype(mm_dtype)
    whh_t = jnp.transpose(whh_p).astype(mm_dtype)
    bias = b_p.reshape(1, G).astype(jnp.float32)
    wfc_t = jnp.transpose(w_fc).astype(mm_dtype)
    bfc = b_fc.reshape(1, C).astype(jnp.float32)

    def _const(shape):
        return pl.BlockSpec(shape, lambda bt, c: (0, 0))

    last = n_chunks - 1
    out = pl.pallas_call(
        _lstm_kernel,
        out_shape=jax.ShapeDtypeStruct((B, C), jnp.float32),
        grid_spec=pltpu.PrefetchScalarGridSpec(
            num_scalar_prefetch=0,
            grid=(2, n_chunks),
            in_specs=[
                pl.BlockSpec((t_chunk, b_tile, D), lambda bt, c: (0, bt, 0)),
                pl.BlockSpec((t_chunk, b_tile, D),
                             lambda bt, c: (jnp.minimum(c + 1, last), bt, 0)),
                _const((D, G)),
                _const((H, G)),
                _const((1, G)),
                _const((H, C)),
                _const((1, C)),
            ],
            out_specs=pl.BlockSpec((b_tile, C), lambda bt, c: (bt, 0)),
            scratch_shapes=[
                pltpu.VMEM((b_tile, H), jnp.float32),
                pltpu.VMEM((b_tile, H), jnp.float32),
                pltpu.VMEM((t_chunk, b_tile, G), jnp.bfloat16),
                pltpu.VMEM((t_chunk, b_tile, G), jnp.bfloat16),
                pltpu.VMEM((b_tile, G), jnp.bfloat16),
            ],
        ),
        compiler_params=pltpu.CompilerParams(
            dimension_semantics=("parallel", "arbitrary"),
            vmem_limit_bytes=100 * 1024 * 1024,
        ),
        cost_estimate=pl.CostEstimate(
            flops=2 * T * B * (D + H) * G + 2 * B * H * C,
            transcendentals=5 * T * B * H,
            bytes_accessed=B * T * D * 2 + (D + H) * G * 2 + B * C * 4,
        ),
    )(x_tm, x_tm, wih_t, whh_t, bias, wfc_t, bfc)

    return out
